# Initial kernel scaffold; baseline (speedup 1.0000x reference)
import jax
import jax.numpy as jnp
from jax.experimental import pallas as pl


def kernel(x, edge_index, edge_attr, u, batch, W1, b1, W2, b2, W3, b3):
    raise NotImplementedError("write your pallas kernel here")


# speedup vs baseline: 11.3696x; 11.3696x over previous
import functools

import jax
import jax.numpy as jnp
from jax.experimental import pallas as pl
from jax.experimental.pallas import tpu as pltpu

N_NODES = 10000
D_FEAT = 256
N_GRAPHS = 128
D_GLOBAL = 64
HIDDEN = 256
OUT = 256

BLK = 2000
GRID = N_NODES // BLK


def _tc_body(x_ref, batch_ref, u_ref, w1u_ref, w1m_ref, b1_ref, w2_ref,
             b2_ref, w3_ref, b3_ref, out_ref, acc_ref, cnt_ref):
    i = pl.program_id(0)

    batch_col = batch_ref[0, 0, :].reshape(BLK, 1)
    seg_ids = jax.lax.broadcasted_iota(jnp.int32, (1, N_GRAPHS), 1)
    onehot = (batch_col == seg_ids).astype(jnp.float32)

    part = jax.lax.dot_general(onehot, x_ref[...],
                               (((0,), (0,)), ((), ())),
                               preferred_element_type=jnp.float32)
    pcnt = jnp.sum(onehot, axis=0, keepdims=True)

    @pl.when(i == 0)
    def _init():
        acc_ref[...] = part
        cnt_ref[...] = pcnt

    @pl.when(i > 0)
    def _accum():
        acc_ref[...] += part
        cnt_ref[...] += pcnt

    @pl.when(i == GRID - 1)
    def _finish():
        counts = jnp.maximum(cnt_ref[0, :], 1.0)
        mean = acc_ref[...] * (1.0 / counts)[:, None]
        h = u_ref[...] @ w1u_ref[...] + mean @ w1m_ref[...] + b1_ref[...]
        h = jnp.maximum(h, 0.0)
        h = jnp.maximum(h @ w2_ref[...] + b2_ref[...], 0.0)
        out_ref[...] = h @ w3_ref[...] + b3_ref[...]


@functools.partial(jax.jit, static_argnames=())
def _run(x, batch3, u, W1u, W1m, b1, W2, b2, W3, b3):
    return pl.pallas_call(
        _tc_body,
        grid=(GRID,),
        in_specs=[
            pl.BlockSpec((BLK, D_FEAT), lambda i: (i, 0)),
            pl.BlockSpec((1, 1, BLK), lambda i: (i, 0, 0)),
            pl.BlockSpec((N_GRAPHS, D_GLOBAL), lambda i: (0, 0)),
            pl.BlockSpec((D_GLOBAL, HIDDEN), lambda i: (0, 0)),
            pl.BlockSpec((D_FEAT, HIDDEN), lambda i: (0, 0)),
            pl.BlockSpec((1, HIDDEN), lambda i: (0, 0)),
            pl.BlockSpec((HIDDEN, HIDDEN), lambda i: (0, 0)),
            pl.BlockSpec((1, HIDDEN), lambda i: (0, 0)),
            pl.BlockSpec((HIDDEN, OUT), lambda i: (0, 0)),
            pl.BlockSpec((1, OUT), lambda i: (0, 0)),
        ],
        out_specs=pl.BlockSpec((N_GRAPHS, OUT), lambda i: (0, 0)),
        out_shape=jax.ShapeDtypeStruct((N_GRAPHS, OUT), jnp.float32),
        scratch_shapes=[
            pltpu.VMEM((N_GRAPHS, D_FEAT), jnp.float32),
            pltpu.VMEM((1, N_GRAPHS), jnp.float32),
        ],
        compiler_params=pltpu.CompilerParams(
            dimension_semantics=("arbitrary",),
        ),
    )(x, batch3, u, W1u, W1m, b1, W2, b2, W3, b3)


def kernel(x, edge_index, edge_attr, u, batch, W1, b1, W2, b2, W3, b3):
    del edge_index, edge_attr
    batch3 = batch.reshape(GRID, 1, BLK)
    W1u = W1[:D_GLOBAL]
    W1m = W1[D_GLOBAL:]
    return _run(x, batch3, u, W1u, W1m, b1.reshape(1, -1), W2,
                b2.reshape(1, -1), W3, b3.reshape(1, -1))
